# baseline (speedup 1.0000x reference)
import jax
import jax.numpy as jnp
from jax.experimental import pallas as pl
from jax.experimental.pallas import tpu as pltpu


def _gcn_kernel(x_ref, w_ref, b_ref, adj_a_ref, adj_b_ref, o_ref, s_ref):
    @pl.when(pl.program_id(0) == 0)
    def _():
        s_ref[...] = jnp.dot(x_ref[...], w_ref[...],
                             preferred_element_type=jnp.float32)

    half = adj_a_ref.shape[0]
    acc_a = jnp.dot(adj_a_ref[...], s_ref[...],
                    preferred_element_type=jnp.float32)
    o_ref[0:half, :] = jnp.maximum(acc_a + b_ref[...], 0.0)
    acc_b = jnp.dot(adj_b_ref[...], s_ref[...],
                    preferred_element_type=jnp.float32)
    o_ref[half:2 * half, :] = jnp.maximum(acc_b + b_ref[...], 0.0)


def kernel(x, adj, W, b):
    n_rows, d_in = x.shape
    d_out = W.shape[1]
    n_cols = adj.shape[1]

    bm = 400
    half = bm // 2
    b2 = b.reshape(1, d_out)
    out = pl.pallas_call(
        _gcn_kernel,
        grid=(pl.cdiv(n_rows, bm),),
        in_specs=[
            pl.BlockSpec((n_rows, d_in), lambda m: (0, 0)),
            pl.BlockSpec((d_in, d_out), lambda m: (0, 0)),
            pl.BlockSpec((1, d_out), lambda m: (0, 0)),
            pl.BlockSpec((half, n_cols), lambda m: (2 * m, 0)),
            pl.BlockSpec((half, n_cols), lambda m: (2 * m + 1, 0)),
        ],
        out_specs=pl.BlockSpec((bm, d_out), lambda m: (m, 0)),
        out_shape=jax.ShapeDtypeStruct((n_rows, d_out), jnp.float32),
        scratch_shapes=[pltpu.VMEM((n_cols, d_out), jnp.float32)],
    )(x, W, b2, adj, adj)
    return out


# speedup vs baseline: 1.0171x; 1.0171x over previous
import jax
import jax.numpy as jnp
from jax.experimental import pallas as pl
from jax.experimental.pallas import tpu as pltpu


def _gcn_kernel(x_ref, w_ref, b_ref, adj_ref, o_ref, s_ref):
    @pl.when(pl.program_id(0) == 0)
    def _():
        s_ref[...] = jnp.dot(x_ref[...], w_ref[...],
                             preferred_element_type=jnp.float32)

    m = pl.program_id(0)
    bm = adj_ref.shape[0]
    acc = jnp.dot(adj_ref[...], s_ref[...],
                  preferred_element_type=jnp.float32)
    o_ref[pl.ds(m * bm, bm), :] = jnp.maximum(acc + b_ref[...], 0.0)


def kernel(x, adj, W, b):
    n_rows, d_in = x.shape
    d_out = W.shape[1]
    n_cols = adj.shape[1]

    bm = 400
    b2 = b.reshape(1, d_out)
    out = pl.pallas_call(
        _gcn_kernel,
        grid=(pl.cdiv(n_rows, bm),),
        in_specs=[
            pl.BlockSpec((n_rows, d_in), lambda m: (0, 0)),
            pl.BlockSpec((d_in, d_out), lambda m: (0, 0)),
            pl.BlockSpec((1, d_out), lambda m: (0, 0)),
            pl.BlockSpec((bm, n_cols), lambda m: (m, 0)),
        ],
        out_specs=pl.BlockSpec((n_rows, d_out), lambda m: (0, 0)),
        out_shape=jax.ShapeDtypeStruct((n_rows, d_out), jnp.float32),
        scratch_shapes=[pltpu.VMEM((n_cols, d_out), jnp.float32)],
    )(x, W, b2, adj)
    return out
